# baseline (speedup 1.0000x reference)
import functools

import jax
import jax.numpy as jnp
from jax import lax
from jax.experimental import pallas as pl
from jax.experimental.pallas import tpu as pltpu
from jax.experimental.pallas import tpu_sc as plsc

BATCH = 16384
D = 16
V_ID = 100001
V_GICS = 1001
LANES = 16


def _build(nc, ns):
    mesh = plsc.VectorSubcoreMesh(core_axis_name="c", subcore_axis_name="s")

    @functools.partial(
        pl.kernel,
        mesh=mesh,
        compiler_params=pltpu.CompilerParams(
            use_tc_tiling_on_sc=True, needs_layout_passes=False),
        out_type=jax.ShapeDtypeStruct((2 * D, BATCH), jnp.int32),
        scratch_types=[
            pltpu.VMEM((V_ID,), jnp.float32),
            pltpu.VMEM((V_GICS,), jnp.float32),
            pltpu.VMEM((BATCH,), jnp.int32),
            pltpu.SemaphoreType.DMA,
            pltpu.SemaphoreType.DMA,
        ],
    )
    def k(id_hbm, gics_hbm, tid_hbm, tgics_hbm, out_hbm,
          src_id, src_gics, idx_v, s_tab, s_idx):
        wid = lax.axis_index("s") * nc + lax.axis_index("c")

        def gather_inplace(src_ref):
            @plsc.parallel_loop(0, BATCH, LANES, unroll=8)
            def _gather(g):
                iv = idx_v[pl.ds(g, LANES)]
                vals = plsc.load_gather(src_ref, [iv])
                idx_v[pl.ds(g, LANES)] = plsc.bitcast(vals, jnp.int32)

        @pl.when(wid < D)
        def _():
            cp_t = pltpu.async_copy(tid_hbm.at[wid], src_id, s_tab)
            cp_i = pltpu.async_copy(id_hbm, idx_v, s_idx)
            cp_t.wait()
            cp_i.wait()
            gather_inplace(src_id)

        @pl.when(wid >= D)
        def _():
            cp_t = pltpu.async_copy(tgics_hbm.at[wid - D], src_gics, s_tab)
            cp_i = pltpu.async_copy(gics_hbm, idx_v, s_idx)
            cp_t.wait()
            cp_i.wait()
            gather_inplace(src_gics)

        pltpu.sync_copy(idx_v, out_hbm.at[wid])

    return k


def kernel(item_id, item_gics, table_item_id, table_item_gics):
    info = plsc.get_sparse_core_info()
    k = _build(info.num_cores, info.num_subcores)
    out_t = k(item_id, item_gics, table_item_id.T, table_item_gics.T)
    return lax.bitcast_convert_type(out_t, jnp.float32).T


# speedup vs baseline: 1.0777x; 1.0777x over previous
import functools

import jax
import jax.numpy as jnp
from jax import lax
from jax.experimental import pallas as pl
from jax.experimental.pallas import tpu as pltpu
from jax.experimental.pallas import tpu_sc as plsc

BATCH = 16384
D = 16
V_ID = 100001
V_GICS = 1001
LANES = 16
CHUNK = 4096
NCHUNK = BATCH // CHUNK


def _build(nc, ns):
    mesh = plsc.VectorSubcoreMesh(core_axis_name="c", subcore_axis_name="s")

    @functools.partial(
        pl.kernel,
        mesh=mesh,
        compiler_params=pltpu.CompilerParams(
            use_tc_tiling_on_sc=True, needs_layout_passes=False),
        out_type=jax.ShapeDtypeStruct((2 * D, BATCH), jnp.float32),
        scratch_types=[
            pltpu.VMEM((V_ID,), jnp.float32),
            pltpu.VMEM((V_GICS,), jnp.float32),
            pltpu.VMEM((BATCH,), jnp.int32),
            pltpu.VMEM((2, CHUNK), jnp.float32),
            pltpu.SemaphoreType.DMA,
            pltpu.SemaphoreType.DMA,
            pltpu.SemaphoreType.DMA,
            pltpu.SemaphoreType.DMA,
        ],
    )
    def k(id_hbm, gics_hbm, tid_hbm, tgics_hbm, out_hbm,
          src_id, src_gics, idx_v, out_v, s_tab, s_idx, s_o0, s_o1):
        wid = lax.axis_index("s") * nc + lax.axis_index("c")
        s_out = (s_o0, s_o1)

        def gather_all(src_ref):
            out_cp = [None, None]
            for c in range(NCHUNK):
                buf = c % 2
                if out_cp[buf] is not None:
                    out_cp[buf].wait()

                @plsc.parallel_loop(0, CHUNK, LANES, unroll=8)
                def _gather(g):
                    iv = idx_v[pl.ds(c * CHUNK + g, LANES)]
                    out_v[buf, pl.ds(g, LANES)] = plsc.load_gather(
                        src_ref, [iv])

                out_cp[buf] = pltpu.async_copy(
                    out_v.at[buf], out_hbm.at[wid, pl.ds(c * CHUNK, CHUNK)],
                    s_out[buf])
            for buf in range(2):
                out_cp[buf].wait()

        @pl.when(wid < D)
        def _():
            cp_t = pltpu.async_copy(tid_hbm.at[wid], src_id, s_tab)
            cp_i = pltpu.async_copy(id_hbm, idx_v, s_idx)
            cp_t.wait()
            cp_i.wait()
            gather_all(src_id)

        @pl.when(wid >= D)
        def _():
            cp_t = pltpu.async_copy(tgics_hbm.at[wid - D], src_gics, s_tab)
            cp_i = pltpu.async_copy(gics_hbm, idx_v, s_idx)
            cp_t.wait()
            cp_i.wait()
            gather_all(src_gics)

    return k


def kernel(item_id, item_gics, table_item_id, table_item_gics):
    info = plsc.get_sparse_core_info()
    k = _build(info.num_cores, info.num_subcores)
    out_t = k(item_id, item_gics, table_item_id.T, table_item_gics.T)
    return out_t.T


# speedup vs baseline: 1.0792x; 1.0014x over previous
import functools

import jax
import jax.numpy as jnp
from jax import lax
from jax.experimental import pallas as pl
from jax.experimental.pallas import tpu as pltpu
from jax.experimental.pallas import tpu_sc as plsc

BATCH = 16384
D = 16
V_ID = 100001
V_GICS = 1001
LANES = 16
CHUNK = 4096
NCHUNK = BATCH // CHUNK


def _build(nc, ns):
    mesh = plsc.VectorSubcoreMesh(core_axis_name="c", subcore_axis_name="s")

    @functools.partial(
        pl.kernel,
        mesh=mesh,
        compiler_params=pltpu.CompilerParams(
            use_tc_tiling_on_sc=True, needs_layout_passes=False,
            skip_device_barrier=True),
        out_type=jax.ShapeDtypeStruct((2 * D, BATCH), jnp.float32),
        scratch_types=[
            pltpu.VMEM((V_ID,), jnp.float32),
            pltpu.VMEM((V_GICS,), jnp.float32),
            pltpu.VMEM((BATCH,), jnp.int32),
            pltpu.VMEM((2, CHUNK), jnp.float32),
            pltpu.SemaphoreType.DMA,
            pltpu.SemaphoreType.DMA,
            pltpu.SemaphoreType.DMA,
            pltpu.SemaphoreType.DMA,
        ],
    )
    def k(id_hbm, gics_hbm, tid_hbm, tgics_hbm, out_hbm,
          src_id, src_gics, idx_v, out_v, s_tab, s_idx, s_o0, s_o1):
        wid = lax.axis_index("s") * nc + lax.axis_index("c")
        s_out = (s_o0, s_o1)

        def gather_all(src_ref):
            out_cp = [None, None]
            for c in range(NCHUNK):
                buf = c % 2
                if out_cp[buf] is not None:
                    out_cp[buf].wait()

                @plsc.parallel_loop(0, CHUNK, LANES, unroll=8)
                def _gather(g):
                    iv = idx_v[pl.ds(c * CHUNK + g, LANES)]
                    out_v[buf, pl.ds(g, LANES)] = plsc.load_gather(
                        src_ref, [iv])

                out_cp[buf] = pltpu.async_copy(
                    out_v.at[buf], out_hbm.at[wid, pl.ds(c * CHUNK, CHUNK)],
                    s_out[buf])
            for buf in range(2):
                out_cp[buf].wait()

        @pl.when(wid < D)
        def _():
            cp_t = pltpu.async_copy(tid_hbm.at[wid], src_id, s_tab)
            cp_i = pltpu.async_copy(id_hbm, idx_v, s_idx)
            cp_t.wait()
            cp_i.wait()
            gather_all(src_id)

        @pl.when(wid >= D)
        def _():
            cp_t = pltpu.async_copy(tgics_hbm.at[wid - D], src_gics, s_tab)
            cp_i = pltpu.async_copy(gics_hbm, idx_v, s_idx)
            cp_t.wait()
            cp_i.wait()
            gather_all(src_gics)

    return k


def kernel(item_id, item_gics, table_item_id, table_item_gics):
    info = plsc.get_sparse_core_info()
    k = _build(info.num_cores, info.num_subcores)
    out_t = k(item_id, item_gics, table_item_id.T, table_item_gics.T)
    return out_t.T
